# baseline (speedup 1.0000x reference)
import functools

import jax
import jax.numpy as jnp
from jax import lax
from jax.experimental import pallas as pl
from jax.experimental.pallas import tpu as pltpu
from jax.experimental.pallas import tpu_sc as plsc

_D = 64
_B = 16384
_NC = 2
_NS = 16
_NW = _NC * _NS
_BPW = _B // _NW
_CH = 128
_NCH = _BPW // _CH


def _gather_call(batch2, weight):
    mesh = plsc.VectorSubcoreMesh(core_axis_name="c", subcore_axis_name="s")

    @functools.partial(
        pl.kernel,
        mesh=mesh,
        out_type=jax.ShapeDtypeStruct((_B, _D), jnp.float32),
        scratch_types=[
            pltpu.VMEM((_NCH, _CH), jnp.int32),
            pltpu.VMEM((_BPW, _D), jnp.float32),
            pltpu.SemaphoreType.DMA,
        ],
        compiler_params=pltpu.CompilerParams(use_tc_tiling_on_sc=False),
    )
    def k(idx_hbm, table_hbm, out_hbm, idx_v, rows_v, sem):
        wid = lax.axis_index("s") * _NC + lax.axis_index("c")
        pltpu.sync_copy(idx_hbm.at[wid], idx_v)
        copies = []
        for j in range(_NCH):
            copies.append(pltpu.async_copy(
                table_hbm.at[idx_v.at[j]],
                rows_v.at[pl.ds(j * _CH, _CH)],
                sem,
            ))
        for c in copies:
            c.wait()
        pltpu.sync_copy(rows_v, out_hbm.at[pl.ds(wid * _BPW, _BPW)])

    return k(batch2, weight)


def kernel(batch, weight):
    batch2 = batch.astype(jnp.int32).reshape(_NW, _NCH, _CH)
    return _gather_call(batch2, weight)


# speedup vs baseline: 1.7067x; 1.7067x over previous
import functools

import jax
import jax.numpy as jnp
from jax import lax
from jax.experimental import pallas as pl
from jax.experimental.pallas import tpu as pltpu
from jax.experimental.pallas import tpu_sc as plsc

_D = 64
_B = 16384
_NC = 2
_NS = 16
_NW = _NC * _NS
_BPW = _B // _NW
_G = 16
_NG = _BPW // _G
_LAG = 3


def _gather_call(bidx, weight):
    mesh = plsc.VectorSubcoreMesh(core_axis_name="c", subcore_axis_name="s")

    @functools.partial(
        pl.kernel,
        mesh=mesh,
        out_type=jax.ShapeDtypeStruct((_B, _D), jnp.float32),
        scratch_types=[
            pltpu.VMEM((_BPW,), jnp.int32),
            pltpu.VMEM((_BPW, _D), jnp.float32),
            pltpu.SemaphoreType.DMA,
        ],
    )
    def k(bidx_hbm, table_hbm, out_hbm, bidx_s, stage, sem):
        wid = lax.axis_index("s") * _NC + lax.axis_index("c")
        pltpu.sync_copy(bidx_hbm.at[wid], bidx_s)

        def fire_group(g):
            base = g * _G
            vec = bidx_s[pl.ds(base, _G)]
            for j in range(_G):
                pltpu.async_copy(
                    table_hbm.at[vec[j]], stage.at[base + j], sem)

        def drain_group():
            pltpu.make_async_copy(
                out_hbm.at[pl.ds(0, _G)], stage.at[pl.ds(0, _G)], sem
            ).wait()

        def body(g, _):
            fire_group(g)

            @pl.when(g >= _LAG)
            def _():
                drain_group()
            return 0

        lax.fori_loop(0, _NG, body, 0)
        for _ in range(_LAG):
            drain_group()

        pltpu.sync_copy(stage, out_hbm.at[pl.ds(wid * _BPW, _BPW)])

    return k(bidx, weight)


def kernel(batch, weight):
    bidx = batch.astype(jnp.int32).reshape(_NW, _BPW)
    return _gather_call(bidx, weight)


# speedup vs baseline: 1.7243x; 1.0103x over previous
import functools

import jax
import jax.numpy as jnp
from jax import lax
from jax.experimental import pallas as pl
from jax.experimental.pallas import tpu as pltpu
from jax.experimental.pallas import tpu_sc as plsc

_D = 64
_B = 16384
_NC = 2
_NS = 16
_NW = _NC * _NS
_BPW = _B // _NW
_G = 16
_NG = _BPW // _G
_LAG = 12


def _gather_call(bidx, weight):
    mesh = plsc.VectorSubcoreMesh(core_axis_name="c", subcore_axis_name="s")

    @functools.partial(
        pl.kernel,
        mesh=mesh,
        out_type=jax.ShapeDtypeStruct((_B, _D), jnp.float32),
        scratch_types=[
            pltpu.VMEM((_BPW,), jnp.int32),
            pltpu.VMEM((_BPW, _D), jnp.float32),
            pltpu.SemaphoreType.DMA,
        ],
    )
    def k(bidx_hbm, table_hbm, out_hbm, bidx_s, stage, sem):
        wid = lax.axis_index("s") * _NC + lax.axis_index("c")
        pltpu.sync_copy(bidx_hbm.at[wid], bidx_s)

        def fire_group(g):
            base = g * _G
            vec = bidx_s[pl.ds(base, _G)]
            for j in range(_G):
                pltpu.async_copy(
                    table_hbm.at[vec[j]], stage.at[base + j], sem)

        def drain_group():
            pltpu.make_async_copy(
                out_hbm.at[pl.ds(0, _G)], stage.at[pl.ds(0, _G)], sem
            ).wait()

        def body(g, _):
            fire_group(g)

            @pl.when(g >= _LAG)
            def _():
                drain_group()
            return 0

        lax.fori_loop(0, _NG, body, 0)
        for _ in range(_LAG):
            drain_group()

        pltpu.sync_copy(stage, out_hbm.at[pl.ds(wid * _BPW, _BPW)])

    return k(bidx, weight)


def kernel(batch, weight):
    bidx = batch.astype(jnp.int32).reshape(_NW, _BPW)
    return _gather_call(bidx, weight)
